# Initial kernel scaffold; baseline (speedup 1.0000x reference)
import jax
import jax.numpy as jnp
from jax.experimental import pallas as pl


def kernel(seq, query, target, embed_table, in_proj_w, in_proj_b, attn_out_w, attn_out_b, ff1_w, ff1_b, ff2_w, ff2_b, ln1_w, ln1_b, ln2_w, ln2_b, gate_w, gate_b, query_embed, qproj_w, qproj_b, rout_w, rout_b):
    raise NotImplementedError("write your pallas kernel here")


# speedup vs baseline: 1.0226x; 1.0226x over previous
import math

import jax
import jax.numpy as jnp
from jax.experimental import pallas as pl
from jax.experimental.pallas import tpu as pltpu

_H = 64
_L = 512
_B = 128
_SLOTS = 6
_V = 64
_DH = 32
_BB = 8
_NPROG = _B // _BB


def _ln(x, w, b):
    mu = jnp.mean(x, axis=1, keepdims=True)
    var = jnp.mean((x - mu) * (x - mu), axis=1, keepdims=True)
    return (x - mu) * jax.lax.rsqrt(var + 1e-5) * w + b


def _fused_kernel(
    seq_ref, query_ref, target_ref, embed_ref,
    wq0_ref, wq1_ref, wk0_ref, wk1_ref, wv0_ref, wv1_ref,
    bq0_ref, bq1_ref, bk0_ref, bk1_ref, bv0_ref, bv1_ref,
    ao0_ref, ao1_ref, aob_ref,
    ff1w_ref, ff1b_ref, ff2w_ref, ff2b_ref,
    ln1w_ref, ln1b_ref, ln2w_ref, ln2b_ref,
    gatew_ref, gateb_ref,
    qemb_ref, qpw_ref, qpb_ref, routw_ref, routb_ref,
    out_ref,
    h2_s, qr_s, tgt_s,
):
    f32 = jnp.float32

    iota_bb = jax.lax.broadcasted_iota(jnp.int32, (_BB, _V), 1)
    qoh = (iota_bb == query_ref[:, :]).astype(f32)
    qh_e = jnp.dot(qoh, qemb_ref[:, :], preferred_element_type=f32)
    qr_s[:, :] = jnp.dot(qh_e, qpw_ref[:, :], preferred_element_type=f32) + qpb_ref[:, :]
    tgt_s[:, :] = (iota_bb == target_ref[:, :]).astype(f32)

    inv_dh = 1.0 / math.sqrt(float(_DH))
    inv_h = 1.0 / math.sqrt(float(_H))
    iota_tok = jax.lax.broadcasted_iota(jnp.int32, (_L, _V), 1)
    iota_col = jax.lax.broadcasted_iota(jnp.int32, (_L, 1), 0)

    wq = (wq0_ref, wq1_ref)
    wk = (wk0_ref, wk1_ref)
    wv = (wv0_ref, wv1_ref)
    bq = (bq0_ref, bq1_ref)
    bk = (bk0_ref, bk1_ref)
    bv = (bv0_ref, bv1_ref)
    ao = (ao0_ref, ao1_ref)

    def row_body(r, acc):
        base = r * _L
        tok = seq_ref[pl.ds(base, _L), :]
        oh = (iota_tok == tok).astype(f32)
        h = jnp.dot(oh, embed_ref[:, :], preferred_element_type=f32)

        attn = aob_ref[:, :]
        for i in range(2):
            qh = (jnp.dot(h, wq[i][:, :], preferred_element_type=f32) + bq[i][:, :]) * inv_dh
            kh = jnp.dot(h, wk[i][:, :], preferred_element_type=f32) + bk[i][:, :]
            vh = jnp.dot(h, wv[i][:, :], preferred_element_type=f32) + bv[i][:, :]
            lg = jax.lax.dot_general(qh, kh, (((1,), (1,)), ((), ())),
                                     preferred_element_type=f32)
            lg = lg - jnp.max(lg, axis=1, keepdims=True)
            p = jnp.exp(lg)
            att = p * (1.0 / jnp.sum(p, axis=1, keepdims=True))
            ah = jnp.dot(att, vh, preferred_element_type=f32)
            attn = attn + jnp.dot(ah, ao[i][:, :], preferred_element_type=f32)

        h1 = _ln(h + attn, ln1w_ref[:, :], ln1b_ref[:, :])
        ffa = jnp.maximum(
            jnp.dot(h1, ff1w_ref[:, :], preferred_element_type=f32) + ff1b_ref[:, :], 0.0)
        ff = jnp.dot(ffa, ff2w_ref[:, :], preferred_element_type=f32) + ff2b_ref[:, :]
        h2 = _ln(h1 + ff, ln2w_ref[:, :], ln2b_ref[:, :])
        h2_s[:, :] = h2

        cur = jnp.sum(h2 * gatew_ref[:, :], axis=1, keepdims=True) + gateb_ref[:, :]

        mem_rows = []
        for _ in range(_SLOTS):
            m = jnp.max(cur)
            idx = jnp.min(jnp.where(cur == m, iota_col, _L))
            mem_rows.append(h2_s[pl.ds(idx, 1), :])
            cur = jnp.where(iota_col == idx, -jnp.inf, cur)
        mem = jnp.concatenate(mem_rows, axis=0)

        qr = qr_s[pl.ds(r, 1), :]
        s = jnp.sum(mem * qr, axis=1, keepdims=True) * inv_h
        s = s - jnp.max(s)
        e = jnp.exp(s)
        w = e * (1.0 / jnp.sum(e))
        pooled = jnp.sum(w * mem, axis=0, keepdims=True)
        logits = jnp.dot(pooled, routw_ref[:, :], preferred_element_type=f32) + routb_ref[:, :]
        mx = jnp.max(logits)
        lse = mx + jnp.log(jnp.sum(jnp.exp(logits - mx)))
        lp = jnp.sum(tgt_s[pl.ds(r, 1), :] * logits) - lse
        return acc - lp

    total = jax.lax.fori_loop(0, _BB, row_body, jnp.float32(0.0))
    out_ref[:, :, :] = jnp.full((1, 1, 128), total, f32)


def kernel(seq, query, target, embed_table, in_proj_w, in_proj_b, attn_out_w,
           attn_out_b, ff1_w, ff1_b, ff2_w, ff2_b, ln1_w, ln1_b, ln2_w, ln2_b,
           gate_w, gate_b, query_embed, qproj_w, qproj_b, rout_w, rout_b):
    f32 = jnp.float32
    seq2 = seq.reshape(_B * _L, 1).astype(jnp.int32)
    q2 = query.reshape(_B, 1).astype(jnp.int32)
    t2 = target.reshape(_B, 1).astype(jnp.int32)

    wq0 = in_proj_w[0:32].T
    wq1 = in_proj_w[32:64].T
    wk0 = in_proj_w[64:96].T
    wk1 = in_proj_w[96:128].T
    wv0 = in_proj_w[128:160].T
    wv1 = in_proj_w[160:192].T
    bq0 = in_proj_b[0:32].reshape(1, 32)
    bq1 = in_proj_b[32:64].reshape(1, 32)
    bk0 = in_proj_b[64:96].reshape(1, 32)
    bk1 = in_proj_b[96:128].reshape(1, 32)
    bv0 = in_proj_b[128:160].reshape(1, 32)
    bv1 = in_proj_b[160:192].reshape(1, 32)
    ao0 = attn_out_w[:, 0:32].T
    ao1 = attn_out_w[:, 32:64].T
    aob = attn_out_b.reshape(1, _H)
    ff1wT = ff1_w.T
    ff1b2 = ff1_b.reshape(1, 2 * _H)
    ff2wT = ff2_w.T
    ff2b2 = ff2_b.reshape(1, _H)
    ln1w2 = ln1_w.reshape(1, _H)
    ln1b2 = ln1_b.reshape(1, _H)
    ln2w2 = ln2_w.reshape(1, _H)
    ln2b2 = ln2_b.reshape(1, _H)
    gatew2 = gate_w.reshape(1, _H)
    gateb2 = gate_b.reshape(1, 1)
    qpwT = qproj_w.T
    qpb2 = qproj_b.reshape(1, _H)
    routwT = rout_w.T
    routb2 = rout_b.reshape(1, _V)

    def full_spec(a):
        shp = a.shape
        return pl.BlockSpec(shp, lambda i, _n=len(shp): (0,) * _n)

    operands = [
        seq2, q2, t2, embed_table,
        wq0, wq1, wk0, wk1, wv0, wv1,
        bq0, bq1, bk0, bk1, bv0, bv1,
        ao0, ao1, aob,
        ff1wT, ff1b2, ff2wT, ff2b2,
        ln1w2, ln1b2, ln2w2, ln2b2,
        gatew2, gateb2,
        query_embed, qpwT, qpb2, routwT, routb2,
    ]
    in_specs = [
        pl.BlockSpec((_BB * _L, 1), lambda i: (i, 0)),
        pl.BlockSpec((_BB, 1), lambda i: (i, 0)),
        pl.BlockSpec((_BB, 1), lambda i: (i, 0)),
    ] + [full_spec(a) for a in operands[3:]]

    partial = pl.pallas_call(
        _fused_kernel,
        grid=(_NPROG,),
        in_specs=in_specs,
        out_specs=pl.BlockSpec((1, 1, 128), lambda i: (i, 0, 0)),
        out_shape=jax.ShapeDtypeStruct((_NPROG, 1, 128), f32),
        scratch_shapes=[
            pltpu.VMEM((_L, _H), f32),
            pltpu.VMEM((_BB, _H), f32),
            pltpu.VMEM((_BB, _H), f32),
        ],
        compiler_params=pltpu.CompilerParams(
            dimension_semantics=("parallel",),
        ),
    )(*operands)

    return jnp.sum(partial[:, 0, 0]) * (1.0 / _B)


# speedup vs baseline: 1.1837x; 1.1575x over previous
import math

import jax
import jax.numpy as jnp
from jax.experimental import pallas as pl
from jax.experimental.pallas import tpu as pltpu

_H = 64
_L = 512
_B = 128
_SLOTS = 6
_V = 64
_DH = 32
_BB = 8
_NPROG = _B // _BB


def _ln(x, w, b):
    mu = jnp.mean(x, axis=1, keepdims=True)
    var = jnp.mean((x - mu) * (x - mu), axis=1, keepdims=True)
    return (x - mu) * jax.lax.rsqrt(var + 1e-5) * w + b


def _fused_kernel(
    seq_ref, query_ref, target_ref, embed_ref,
    wq0_ref, wq1_ref, wk0_ref, wk1_ref, wv0_ref, wv1_ref,
    bq0_ref, bq1_ref, bk0_ref, bk1_ref, bv0_ref, bv1_ref,
    ao0_ref, ao1_ref, aob_ref,
    ff1w_ref, ff1b_ref, ff2w_ref, ff2b_ref,
    ln1w_ref, ln1b_ref, ln2w_ref, ln2b_ref,
    gatew_ref, gateb_ref,
    qemb_ref, qpw_ref, qpb_ref, routw_ref, routb_ref,
    out_ref,
    qr_s, tgt_s,
):
    f32 = jnp.float32

    iota_bb = jax.lax.broadcasted_iota(jnp.int32, (_BB, _V), 1)
    qoh = (iota_bb == query_ref[:, :]).astype(f32)
    qh_e = jnp.dot(qoh, qemb_ref[:, :], preferred_element_type=f32)
    qr_s[:, :] = jnp.dot(qh_e, qpw_ref[:, :], preferred_element_type=f32) + qpb_ref[:, :]
    tgt_s[:, :] = (iota_bb == target_ref[:, :]).astype(f32)

    inv_dh = 1.0 / math.sqrt(float(_DH))
    inv_h = 1.0 / math.sqrt(float(_H))
    iota_tok = jax.lax.broadcasted_iota(jnp.int32, (_L, _V), 1)
    iota_row = jax.lax.broadcasted_iota(jnp.int32, (1, _L), 1)

    wq = (wq0_ref, wq1_ref)
    wk = (wk0_ref, wk1_ref)
    wv = (wv0_ref, wv1_ref)
    bq = (bq0_ref, bq1_ref)
    bk = (bk0_ref, bk1_ref)
    bv = (bv0_ref, bv1_ref)
    ao = (ao0_ref, ao1_ref)

    def row_compute(r):
        base = r * _L
        tok = seq_ref[pl.ds(base, _L), :]
        oh = (iota_tok == tok).astype(f32)
        h = jnp.dot(oh, embed_ref[:, :], preferred_element_type=f32)

        attn = aob_ref[:, :]
        for i in range(2):
            qh = (jnp.dot(h, wq[i][:, :], preferred_element_type=f32) + bq[i][:, :]) * inv_dh
            kh = jnp.dot(h, wk[i][:, :], preferred_element_type=f32) + bk[i][:, :]
            vh = jnp.dot(h, wv[i][:, :], preferred_element_type=f32) + bv[i][:, :]
            lg = jax.lax.dot_general(qh, kh, (((1,), (1,)), ((), ())),
                                     preferred_element_type=f32)
            p = jnp.exp(lg - jnp.max(lg, axis=1, keepdims=True))
            ssum = jnp.sum(p, axis=1, keepdims=True)
            ah = jnp.dot(p, vh, preferred_element_type=f32) * (1.0 / ssum)
            attn = attn + jnp.dot(ah, ao[i][:, :], preferred_element_type=f32)

        h1 = _ln(h + attn, ln1w_ref[:, :], ln1b_ref[:, :])
        ffa = jnp.maximum(
            jnp.dot(h1, ff1w_ref[:, :], preferred_element_type=f32) + ff1b_ref[:, :], 0.0)
        ff = jnp.dot(ffa, ff2w_ref[:, :], preferred_element_type=f32) + ff2b_ref[:, :]
        h2 = _ln(h1 + ff, ln2w_ref[:, :], ln2b_ref[:, :])

        qr = qr_s[pl.ds(r, 1), :]
        g_col = jnp.dot(h2, gatew_ref[:, :], preferred_element_type=f32)
        q_col = jnp.dot(h2, jnp.transpose(qr), preferred_element_type=f32)
        g = jnp.transpose(g_col) + gateb_ref[:, :]
        qs = jnp.transpose(q_col) * inv_h

        cur = g
        sel = jnp.zeros((1, _L), jnp.bool_)
        for _ in range(_SLOTS):
            m = jnp.max(cur)
            idx = jnp.min(jnp.where(cur == m, iota_row, _L))
            hit = iota_row == idx
            sel = jnp.logical_or(sel, hit)
            cur = jnp.where(hit, -jnp.inf, cur)

        ms = jnp.max(jnp.where(sel, qs, -jnp.inf))
        e = jnp.where(sel, jnp.exp(qs - ms), 0.0)
        w = e * (1.0 / jnp.sum(e))
        pooled = jnp.dot(w, h2, preferred_element_type=f32)
        logits = jnp.dot(pooled, routw_ref[:, :], preferred_element_type=f32) + routb_ref[:, :]
        mx = jnp.max(logits)
        lse = mx + jnp.log(jnp.sum(jnp.exp(logits - mx)))
        lp = jnp.sum(tgt_s[pl.ds(r, 1), :] * logits) - lse
        return -lp

    def row_body(r, acc):
        return acc + row_compute(r) + row_compute(r + _BB // 2)

    total = jax.lax.fori_loop(0, _BB // 2, row_body, jnp.float32(0.0))
    out_ref[:, :, :] = jnp.full((1, 1, 128), total, f32)


def kernel(seq, query, target, embed_table, in_proj_w, in_proj_b, attn_out_w,
           attn_out_b, ff1_w, ff1_b, ff2_w, ff2_b, ln1_w, ln1_b, ln2_w, ln2_b,
           gate_w, gate_b, query_embed, qproj_w, qproj_b, rout_w, rout_b):
    f32 = jnp.float32
    seq2 = seq.reshape(_B * _L, 1).astype(jnp.int32)
    q2 = query.reshape(_B, 1).astype(jnp.int32)
    t2 = target.reshape(_B, 1).astype(jnp.int32)

    wq0 = in_proj_w[0:32].T
    wq1 = in_proj_w[32:64].T
    wk0 = in_proj_w[64:96].T
    wk1 = in_proj_w[96:128].T
    wv0 = in_proj_w[128:160].T
    wv1 = in_proj_w[160:192].T
    bq0 = in_proj_b[0:32].reshape(1, 32)
    bq1 = in_proj_b[32:64].reshape(1, 32)
    bk0 = in_proj_b[64:96].reshape(1, 32)
    bk1 = in_proj_b[96:128].reshape(1, 32)
    bv0 = in_proj_b[128:160].reshape(1, 32)
    bv1 = in_proj_b[160:192].reshape(1, 32)
    ao0 = attn_out_w[:, 0:32].T
    ao1 = attn_out_w[:, 32:64].T
    aob = attn_out_b.reshape(1, _H)
    ff1wT = ff1_w.T
    ff1b2 = ff1_b.reshape(1, 2 * _H)
    ff2wT = ff2_w.T
    ff2b2 = ff2_b.reshape(1, _H)
    ln1w2 = ln1_w.reshape(1, _H)
    ln1b2 = ln1_b.reshape(1, _H)
    ln2w2 = ln2_w.reshape(1, _H)
    ln2b2 = ln2_b.reshape(1, _H)
    gatew2 = gate_w.reshape(1, _H).T
    gateb2 = gate_b.reshape(1, 1)
    qpwT = qproj_w.T
    qpb2 = qproj_b.reshape(1, _H)
    routwT = rout_w.T
    routb2 = rout_b.reshape(1, _V)

    def full_spec(a):
        shp = a.shape
        return pl.BlockSpec(shp, lambda i, _n=len(shp): (0,) * _n)

    operands = [
        seq2, q2, t2, embed_table,
        wq0, wq1, wk0, wk1, wv0, wv1,
        bq0, bq1, bk0, bk1, bv0, bv1,
        ao0, ao1, aob,
        ff1wT, ff1b2, ff2wT, ff2b2,
        ln1w2, ln1b2, ln2w2, ln2b2,
        gatew2, gateb2,
        query_embed, qpwT, qpb2, routwT, routb2,
    ]
    in_specs = [
        pl.BlockSpec((_BB * _L, 1), lambda i: (i, 0)),
        pl.BlockSpec((_BB, 1), lambda i: (i, 0)),
        pl.BlockSpec((_BB, 1), lambda i: (i, 0)),
    ] + [full_spec(a) for a in operands[3:]]

    partial = pl.pallas_call(
        _fused_kernel,
        grid=(_NPROG,),
        in_specs=in_specs,
        out_specs=pl.BlockSpec((1, 1, 128), lambda i: (i, 0, 0)),
        out_shape=jax.ShapeDtypeStruct((_NPROG, 1, 128), f32),
        scratch_shapes=[
            pltpu.VMEM((_BB, _H), f32),
            pltpu.VMEM((_BB, _H), f32),
        ],
        compiler_params=pltpu.CompilerParams(
            dimension_semantics=("parallel",),
        ),
    )(*operands)

    return jnp.sum(partial[:, 0, 0]) * (1.0 / _B)


# speedup vs baseline: 1.2139x; 1.0255x over previous
import math

import jax
import jax.numpy as jnp
from jax.experimental import pallas as pl
from jax.experimental.pallas import tpu as pltpu

_H = 64
_L = 512
_B = 128
_SLOTS = 6
_V = 64
_DH = 32
_BB = 8
_NPROG = _B // _BB


def _ln(x, w, b):
    mu = jnp.mean(x, axis=1, keepdims=True)
    var = jnp.mean((x - mu) * (x - mu), axis=1, keepdims=True)
    return (x - mu) * jax.lax.rsqrt(var + 1e-5) * w + b


def _fused_kernel(
    seq_ref, query_ref, target_ref, embed_ref,
    wq0_ref, wq1_ref, wk0_ref, wk1_ref, wv0_ref, wv1_ref,
    bq0_ref, bq1_ref, bk0_ref, bk1_ref, bv0_ref, bv1_ref,
    ao0_ref, ao1_ref, aob_ref,
    ff1w_ref, ff1b_ref, ff2w_ref, ff2b_ref,
    ln1w_ref, ln1b_ref, ln2w_ref, ln2b_ref,
    gatew_ref, gateb_ref,
    qemb_ref, qpw_ref, qpb_ref, routw_ref, routb_ref,
    out_ref,
    qr_s, tgt_s,
):
    f32 = jnp.float32

    iota_bb = jax.lax.broadcasted_iota(jnp.int32, (_BB, _V), 1)
    qoh = (iota_bb == query_ref[:, :]).astype(f32)
    qh_e = jnp.dot(qoh, qemb_ref[:, :], preferred_element_type=f32)
    qr_s[:, :] = jnp.dot(qh_e, qpw_ref[:, :], preferred_element_type=f32) + qpb_ref[:, :]
    tgt_s[:, :] = (iota_bb == target_ref[:, :]).astype(f32)

    inv_dh = 1.0 / math.sqrt(float(_DH))
    inv_h = 1.0 / math.sqrt(float(_H))
    iota_tok = jax.lax.broadcasted_iota(jnp.int32, (_L, _V), 1)
    iota_row = jax.lax.broadcasted_iota(jnp.int32, (1, _L), 1)

    wq = (wq0_ref, wq1_ref)
    wk = (wk0_ref, wk1_ref)
    wv = (wv0_ref, wv1_ref)
    bq = (bq0_ref, bq1_ref)
    bk = (bk0_ref, bk1_ref)
    bv = (bv0_ref, bv1_ref)
    ao = (ao0_ref, ao1_ref)

    def row_compute(r):
        base = r * _L
        tok = seq_ref[pl.ds(base, _L), :]
        oh = (iota_tok == tok).astype(f32)
        h = jnp.dot(oh, embed_ref[:, :], preferred_element_type=f32)

        attn = aob_ref[:, :]
        for i in range(2):
            qh = (jnp.dot(h, wq[i][:, :], preferred_element_type=f32) + bq[i][:, :]) * inv_dh
            kh = jnp.dot(h, wk[i][:, :], preferred_element_type=f32) + bk[i][:, :]
            vh = jnp.dot(h, wv[i][:, :], preferred_element_type=f32) + bv[i][:, :]
            lg = jax.lax.dot_general(qh, kh, (((1,), (1,)), ((), ())),
                                     preferred_element_type=f32)
            p = jnp.exp(lg - jnp.max(lg, axis=1, keepdims=True))
            ssum = jnp.sum(p, axis=1, keepdims=True)
            ah = jnp.dot(p, vh, preferred_element_type=f32) * (1.0 / ssum)
            attn = attn + jnp.dot(ah, ao[i][:, :], preferred_element_type=f32)

        h1 = _ln(h + attn, ln1w_ref[:, :], ln1b_ref[:, :])
        ffa = jnp.maximum(
            jnp.dot(h1, ff1w_ref[:, :], preferred_element_type=f32) + ff1b_ref[:, :], 0.0)
        ff = jnp.dot(ffa, ff2w_ref[:, :], preferred_element_type=f32) + ff2b_ref[:, :]
        h2 = _ln(h1 + ff, ln2w_ref[:, :], ln2b_ref[:, :])

        qr = qr_s[pl.ds(r, 1), :]
        g_col = jnp.dot(h2, gatew_ref[:, :], preferred_element_type=f32)
        q_col = jnp.dot(h2, jnp.transpose(qr), preferred_element_type=f32)
        g = jnp.transpose(g_col) + gateb_ref[:, :]
        qs = jnp.transpose(q_col) * inv_h

        cur = g
        sel = jnp.zeros((1, _L), jnp.bool_)
        for _ in range(_SLOTS):
            m = jnp.max(cur)
            idx = jnp.min(jnp.where(cur == m, iota_row, _L))
            hit = iota_row == idx
            sel = jnp.logical_or(sel, hit)
            cur = jnp.where(hit, -jnp.inf, cur)

        ms = jnp.max(jnp.where(sel, qs, -jnp.inf))
        e = jnp.where(sel, jnp.exp(qs - ms), 0.0)
        w = e * (1.0 / jnp.sum(e))
        pooled = jnp.dot(w, h2, preferred_element_type=f32)
        logits = jnp.dot(pooled, routw_ref[:, :], preferred_element_type=f32) + routb_ref[:, :]
        mx = jnp.max(logits)
        lse = mx + jnp.log(jnp.sum(jnp.exp(logits - mx)))
        lp = jnp.sum(tgt_s[pl.ds(r, 1), :] * logits) - lse
        return -lp

    def row_body(r, acc):
        q = _BB // 4
        return (acc + row_compute(r) + row_compute(r + q)
                + row_compute(r + 2 * q) + row_compute(r + 3 * q))

    total = jax.lax.fori_loop(0, _BB // 4, row_body, jnp.float32(0.0))
    out_ref[:, :, :] = jnp.full((1, 1, 128), total, f32)


def kernel(seq, query, target, embed_table, in_proj_w, in_proj_b, attn_out_w,
           attn_out_b, ff1_w, ff1_b, ff2_w, ff2_b, ln1_w, ln1_b, ln2_w, ln2_b,
           gate_w, gate_b, query_embed, qproj_w, qproj_b, rout_w, rout_b):
    f32 = jnp.float32
    seq2 = seq.reshape(_B * _L, 1).astype(jnp.int32)
    q2 = query.reshape(_B, 1).astype(jnp.int32)
    t2 = target.reshape(_B, 1).astype(jnp.int32)

    wq0 = in_proj_w[0:32].T
    wq1 = in_proj_w[32:64].T
    wk0 = in_proj_w[64:96].T
    wk1 = in_proj_w[96:128].T
    wv0 = in_proj_w[128:160].T
    wv1 = in_proj_w[160:192].T
    bq0 = in_proj_b[0:32].reshape(1, 32)
    bq1 = in_proj_b[32:64].reshape(1, 32)
    bk0 = in_proj_b[64:96].reshape(1, 32)
    bk1 = in_proj_b[96:128].reshape(1, 32)
    bv0 = in_proj_b[128:160].reshape(1, 32)
    bv1 = in_proj_b[160:192].reshape(1, 32)
    ao0 = attn_out_w[:, 0:32].T
    ao1 = attn_out_w[:, 32:64].T
    aob = attn_out_b.reshape(1, _H)
    ff1wT = ff1_w.T
    ff1b2 = ff1_b.reshape(1, 2 * _H)
    ff2wT = ff2_w.T
    ff2b2 = ff2_b.reshape(1, _H)
    ln1w2 = ln1_w.reshape(1, _H)
    ln1b2 = ln1_b.reshape(1, _H)
    ln2w2 = ln2_w.reshape(1, _H)
    ln2b2 = ln2_b.reshape(1, _H)
    gatew2 = gate_w.reshape(1, _H).T
    gateb2 = gate_b.reshape(1, 1)
    qpwT = qproj_w.T
    qpb2 = qproj_b.reshape(1, _H)
    routwT = rout_w.T
    routb2 = rout_b.reshape(1, _V)

    def full_spec(a):
        shp = a.shape
        return pl.BlockSpec(shp, lambda i, _n=len(shp): (0,) * _n)

    operands = [
        seq2, q2, t2, embed_table,
        wq0, wq1, wk0, wk1, wv0, wv1,
        bq0, bq1, bk0, bk1, bv0, bv1,
        ao0, ao1, aob,
        ff1wT, ff1b2, ff2wT, ff2b2,
        ln1w2, ln1b2, ln2w2, ln2b2,
        gatew2, gateb2,
        query_embed, qpwT, qpb2, routwT, routb2,
    ]
    in_specs = [
        pl.BlockSpec((_BB * _L, 1), lambda i: (i, 0)),
        pl.BlockSpec((_BB, 1), lambda i: (i, 0)),
        pl.BlockSpec((_BB, 1), lambda i: (i, 0)),
    ] + [full_spec(a) for a in operands[3:]]

    partial = pl.pallas_call(
        _fused_kernel,
        grid=(_NPROG,),
        in_specs=in_specs,
        out_specs=pl.BlockSpec((1, 1, 128), lambda i: (i, 0, 0)),
        out_shape=jax.ShapeDtypeStruct((_NPROG, 1, 128), f32),
        scratch_shapes=[
            pltpu.VMEM((_BB, _H), f32),
            pltpu.VMEM((_BB, _H), f32),
        ],
        compiler_params=pltpu.CompilerParams(
            dimension_semantics=("parallel",),
        ),
    )(*operands)

    return jnp.sum(partial[:, 0, 0]) * (1.0 / _B)
